# baseline (speedup 1.0000x reference)
import functools

import jax
import jax.numpy as jnp
from jax import lax
from jax.experimental import pallas as pl
from jax.experimental.pallas import tpu as pltpu
from jax.experimental.pallas import tpu_sc as plsc

NUM_EMB = 1_000_000
DIM = 64
BATCH = 4096
SEQ_LEN = 200
B_TOTAL = BATCH * SEQ_LEN

NC = 2
NS = 16
NW = NC * NS

IDX_PER_DMA = 128
K = 8
GROUP = K * IDX_PER_DMA
ROWS_PER_W = B_TOTAL // NW
GROUPS = ROWS_PER_W // GROUP
IDX_ROWS = B_TOTAL // IDX_PER_DMA


@functools.partial(
    pl.kernel,
    mesh=plsc.VectorSubcoreMesh(core_axis_name="c", subcore_axis_name="s"),
    compiler_params=pltpu.CompilerParams(use_tc_tiling_on_sc=False),
    out_type=jax.ShapeDtypeStruct((IDX_ROWS, IDX_PER_DMA, DIM), jnp.float32),
    scratch_types=[
        pltpu.VMEM((K, IDX_PER_DMA), jnp.int32),
        pltpu.VMEM((K, IDX_PER_DMA, DIM), jnp.float32),
        pltpu.SemaphoreType.DMA,
    ],
)
def _gather(w_hbm, idx_hbm, out_hbm, idx_v, rows_v, sem):
    wid = lax.axis_index("s") * NC + lax.axis_index("c")
    grp0 = wid * (ROWS_PER_W // GROUP)

    def body(g, carry):
        base = (grp0 + g) * K
        pltpu.sync_copy(idx_hbm.at[pl.ds(base, K)], idx_v)
        copies = [
            pltpu.async_copy(w_hbm.at[idx_v.at[j]], rows_v.at[j], sem)
            for j in range(K)
        ]
        for c in copies:
            c.wait()
        pltpu.sync_copy(rows_v, out_hbm.at[pl.ds(base, K)])
        return carry

    lax.fori_loop(0, GROUPS, body, 0)


def kernel(token_ids, W):
    idx = token_ids.reshape(IDX_ROWS, IDX_PER_DMA).astype(jnp.int32)
    out = _gather(W, idx)
    return out.reshape(BATCH, SEQ_LEN, DIM)


# speedup vs baseline: 1.0077x; 1.0077x over previous
import functools

import jax
import jax.numpy as jnp
from jax import lax
from jax.experimental import pallas as pl
from jax.experimental.pallas import tpu as pltpu
from jax.experimental.pallas import tpu_sc as plsc

NUM_EMB = 1_000_000
DIM = 64
BATCH = 4096
SEQ_LEN = 200
B_TOTAL = BATCH * SEQ_LEN

NC = 2
NS = 16
NW = NC * NS

IDX_PER_DMA = 128
K = 5
GROUP = K * IDX_PER_DMA
ROWS_PER_W = B_TOTAL // NW
GROUPS = ROWS_PER_W // GROUP
IDX_ROWS = B_TOTAL // IDX_PER_DMA
NBUF = 2


@functools.partial(
    pl.kernel,
    mesh=plsc.VectorSubcoreMesh(core_axis_name="c", subcore_axis_name="s"),
    compiler_params=pltpu.CompilerParams(use_tc_tiling_on_sc=False),
    out_type=jax.ShapeDtypeStruct((IDX_ROWS, IDX_PER_DMA, DIM), jnp.float32),
    scratch_types=[
        pltpu.VMEM((NBUF, K, IDX_PER_DMA), jnp.int32),
        pltpu.VMEM((NBUF, K, IDX_PER_DMA, DIM), jnp.float32),
        pltpu.SemaphoreType.DMA,
        pltpu.SemaphoreType.DMA,
    ],
)
def _gather(w_hbm, idx_hbm, out_hbm, idx_v, rows_v, gsem, osem):
    wid = lax.axis_index("s") * NC + lax.axis_index("c")
    grp0 = wid * GROUPS

    def stage(g, b):
        base = (grp0 + g) * K
        pltpu.sync_copy(idx_hbm.at[pl.ds(base, K)], idx_v.at[b])
        for j in range(K):
            pltpu.async_copy(w_hbm.at[idx_v.at[b, j]], rows_v.at[b, j], gsem)

    stage(0, 0)

    def body(g, carry):
        b = lax.rem(g, NBUF)
        nb = lax.rem(g + 1, NBUF)

        @pl.when(g + 1 < GROUPS)
        def _():
            @pl.when(g >= 1)
            def _():
                pltpu.make_async_copy(
                    rows_v.at[nb], out_hbm.at[pl.ds(0, K)], osem
                ).wait()

            stage(g + 1, nb)

        for j in range(K):
            pltpu.make_async_copy(
                w_hbm.at[idx_v.at[b, j]], rows_v.at[b, j], gsem
            ).wait()
        base = (grp0 + g) * K
        pltpu.async_copy(rows_v.at[b], out_hbm.at[pl.ds(base, K)], osem)
        return carry

    lax.fori_loop(0, GROUPS, body, 0)
    pltpu.make_async_copy(rows_v.at[0], out_hbm.at[pl.ds(0, K)], osem).wait()
    pltpu.make_async_copy(rows_v.at[1], out_hbm.at[pl.ds(0, K)], osem).wait()


def kernel(token_ids, W):
    idx = token_ids.reshape(IDX_ROWS, IDX_PER_DMA).astype(jnp.int32)
    out = _gather(W, idx)
    return out.reshape(BATCH, SEQ_LEN, DIM)
